# Initial kernel scaffold; baseline (speedup 1.0000x reference)
import jax
import jax.numpy as jnp
from jax.experimental import pallas as pl


def kernel(xyz, features, query_rv_xyz, query_rv_coords, rv_map):
    raise NotImplementedError("write your pallas kernel here")


# speedup vs baseline: 26.7629x; 26.7629x over previous
import functools

import jax
import jax.numpy as jnp
from jax import lax
from jax.experimental import pallas as pl
from jax.experimental.pallas import tpu as pltpu
from jax.experimental.pallas import tpu_sc as plsc

RADIUS2 = 4.0
NSAMPLE = 32
NCAND = 180
NCP = 192
M = 16384
CFEAT = 16
RV_H, RV_W, PPP = 64, 2048, 4

NCORES, NSUBC = 2, 16
NW = NCORES * NSUBC
QPW = M // NW
G = 16
NGRP = QPW // G
CH = 128
NCH_CAND = (G * NCP) // CH
NCH_SEL = (G * NSAMPLE) // CH
OW = (3 + CFEAT) * NSAMPLE


def _splat(x, dtype=jnp.int32):
    return jnp.full((16,), x, dtype=dtype)


def _vgather(v, idx):
    return v.at[idx].get(mode="promise_in_bounds")


def _sc_body(qx_h, qy_h, qz_h, row_h, col_h, rvf_h, xyzp_h, feat_h, out_h,
             qxv, qyv, qzv, rowv, colv, eidx, cand, cxyz, sel, gfeat, gxyz,
             cntv, outb, sem):
    wid = lax.axis_index("s") * NCORES + lax.axis_index("c")
    qbase = wid * QPW
    qrow = wid * (QPW // CH)
    orow = wid * (QPW * OW // CH)
    iota = jnp.arange(16, dtype=jnp.int32)

    pltpu.sync_copy(qx_h.at[pl.ds(qrow, QPW // CH)], qxv)
    pltpu.sync_copy(qy_h.at[pl.ds(qrow, QPW // CH)], qyv)
    pltpu.sync_copy(qz_h.at[pl.ds(qrow, QPW // CH)], qzv)
    pltpu.sync_copy(row_h.at[pl.ds(qrow, QPW // CH)], rowv)
    pltpu.sync_copy(col_h.at[pl.ds(qrow, QPW // CH)], colv)

    def group_body(g, carry):
        def build_body(i, bc):
            lq = g * G + i
            lqs = _splat(lq)
            lqr, lqc = lqs >> 7, lqs & 127
            rsp = plsc.load_gather(rowv, [lqr, lqc]) & jnp.int32(RV_H - 1)
            csp = plsc.load_gather(colv, [lqr, lqc]) & jnp.int32(RV_W - 1)
            cells = []
            for jj in range(3):
                u = iota + 16 * jj
                oh = u // 9 - 2
                ow = 2 * (u % 9) - 8
                rr = jnp.clip(rsp + oh, 0, RV_H - 1)
                cc = (csp + ow) & jnp.int32(RV_W - 1)
                cells.append(rr * RV_W + cc)
            for jj2 in range(12):
                lidx = iota // 4 + 4 * (jj2 % 4)
                cv = _vgather(cells[jj2 // 4], lidx)
                ev = cv * PPP + (iota & 3)
                p = _splat(i * NCP + 16 * jj2) + iota
                plsc.store_scatter(eidx, [p >> 7, p & 127], ev)
            return bc

        lax.fori_loop(0, G, build_body, 0)

        cps = []
        for j in range(NCH_CAND):
            cp = pltpu.make_async_copy(rvf_h.at[eidx.at[j]], cand.at[j], sem)
            cp.start()
            cps.append(cp)
        for cp in cps:
            cp.wait()

        cps = []
        for j in range(NCH_CAND):
            cp = pltpu.make_async_copy(xyzp_h.at[cand.at[j]], cxyz.at[j], sem)
            cp.start()
            cps.append(cp)
        for cp in cps:
            cp.wait()

        def select_body(i, bc):
            lq = g * G + i
            lqs = _splat(lq)
            lqr, lqc = lqs >> 7, lqs & 127
            xq = plsc.load_gather(qxv, [lqr, lqc])
            yq = plsc.load_gather(qyv, [lqr, lqc])
            zq = plsc.load_gather(qzv, [lqr, lqc])
            z16 = _splat(0)
            cnt = jnp.int32(0)
            for jj in range(12):
                p = _splat(i * NCP + 16 * jj) + iota
                pr, pc = p >> 7, p & 127
                cd = plsc.load_gather(cand, [pr, pc])
                x = plsc.load_gather(cxyz, [pr, pc, z16])
                y = plsc.load_gather(cxyz, [pr, pc, z16 + 1])
                z = plsc.load_gather(cxyz, [pr, pc, z16 + 2])
                dx, dy, dz = x - xq, y - yq, z - zq
                d2 = dx * dx + dy * dy + dz * dz
                val = d2 < RADIUS2
                if jj == 11:
                    val = val & (iota < (NCAND - 16 * 11))
                vi = val.astype(jnp.int32)
                pref = plsc.cumsum(vi)
                rank = cnt + pref - 1
                m = val & (rank < NSAMPLE)
                sp = _splat(i * NSAMPLE) + rank
                plsc.store_scatter(sel, [sp >> 7, sp & 127], cd, mask=m)
                cnt = cnt + jnp.sum(vi)
            sp0 = i * NSAMPLE
            fsv = plsc.load_gather(
                sel, [_splat(0) + (sp0 >> 7), _splat(0) + (sp0 & 127)])
            for h in range(2):
                k = iota + 16 * h
                spk = sp0 + k
                cur = plsc.load_gather(sel, [spk >> 7, spk & 127])
                new = jnp.where(k < cnt, cur, fsv)
                new = jnp.where(cnt > 0, new, 0)
                plsc.store_scatter(sel, [spk >> 7, spk & 127], new)
            plsc.store_scatter(cntv, [_splat(0), _splat(0) + i],
                               _splat(0) + cnt, mask=iota == 0)
            return bc

        lax.fori_loop(0, G, select_body, 0)

        cps = []
        for j in range(NCH_SEL):
            cp = pltpu.make_async_copy(feat_h.at[sel.at[j]], gfeat.at[j], sem)
            cp.start()
            cps.append(cp)
            cp = pltpu.make_async_copy(xyzp_h.at[sel.at[j]], gxyz.at[j], sem)
            cp.start()
            cps.append(cp)
        for cp in cps:
            cp.wait()

        def out_body(i, bc):
            lq = g * G + i
            lqs = _splat(lq)
            lqr, lqc = lqs >> 7, lqs & 127
            xq = plsc.load_gather(qxv, [lqr, lqc])
            yq = plsc.load_gather(qyv, [lqr, lqc])
            zq = plsc.load_gather(qzv, [lqr, lqc])
            cz = plsc.load_gather(cntv, [_splat(0), _splat(0) + i]) > 0
            qs = (xq, yq, zq)
            for c in range(3 + CFEAT):
                for h in range(2):
                    sp = _splat(i * NSAMPLE + 16 * h) + iota
                    sr, sc = sp >> 7, sp & 127
                    if c < 3:
                        v = plsc.load_gather(gxyz, [sr, sc, _splat(c)]) - qs[c]
                    else:
                        v = plsc.load_gather(gfeat, [sr, sc, _splat(c - 3)])
                    v = jnp.where(cz, v, 0.0)
                    po = _splat(i * OW + c * NSAMPLE + 16 * h) + iota
                    plsc.store_scatter(outb, [po >> 7, po & 127], v)
            return bc

        lax.fori_loop(0, G, out_body, 0)

        pltpu.sync_copy(outb, out_h.at[pl.ds(orow + g * (G * OW // CH),
                                             G * OW // CH)])
        return carry

    lax.fori_loop(0, NGRP, group_body, 0)


@jax.jit
def kernel(xyz, features, query_rv_xyz, query_rv_coords, rv_map):
    xyzp = jnp.concatenate(
        [xyz, jnp.zeros((xyz.shape[0], 5), jnp.float32)], axis=1)
    rvf = rv_map.reshape(-1)
    qx = query_rv_xyz[:, 0].reshape(M // CH, CH)
    qy = query_rv_xyz[:, 1].reshape(M // CH, CH)
    qz = query_rv_xyz[:, 2].reshape(M // CH, CH)
    rows = query_rv_coords[:, 1].reshape(M // CH, CH)
    cols = query_rv_coords[:, 2].reshape(M // CH, CH)

    mesh = plsc.VectorSubcoreMesh(core_axis_name="c", subcore_axis_name="s",
                                  num_cores=NCORES, num_subcores=NSUBC)
    run = pl.kernel(
        _sc_body,
        out_type=jax.ShapeDtypeStruct((M * OW // CH, CH), jnp.float32),
        mesh=mesh,
        compiler_params=pltpu.CompilerParams(use_tc_tiling_on_sc=False,
                                             needs_layout_passes=False),
        scratch_types=[
            pltpu.VMEM((QPW // CH, CH), jnp.float32),
            pltpu.VMEM((QPW // CH, CH), jnp.float32),
            pltpu.VMEM((QPW // CH, CH), jnp.float32),
            pltpu.VMEM((QPW // CH, CH), jnp.int32),
            pltpu.VMEM((QPW // CH, CH), jnp.int32),
            pltpu.VMEM((NCH_CAND, CH), jnp.int32),
            pltpu.VMEM((NCH_CAND, CH), jnp.int32),
            pltpu.VMEM((NCH_CAND, CH, 8), jnp.float32),
            pltpu.VMEM((NCH_SEL, CH), jnp.int32),
            pltpu.VMEM((NCH_SEL, CH, CFEAT), jnp.float32),
            pltpu.VMEM((NCH_SEL, CH, 8), jnp.float32),
            pltpu.VMEM((1, CH), jnp.int32),
            pltpu.VMEM((G * (3 + CFEAT) * NSAMPLE // CH, CH), jnp.float32),
            pltpu.SemaphoreType.DMA,
        ],
    )
    out = run(qx, qy, qz, rows, cols, rvf, xyzp, features)
    return out.reshape(M, 3 + CFEAT, NSAMPLE)
